# baseline (speedup 1.0000x reference)
import jax
import jax.numpy as jnp
from jax.experimental import pallas as pl
from jax.experimental.pallas import tpu as pltpu

_NUM_CONCEPTS = 32
_TOPK = 5
_LANES = 128


def _knn_vote_body(q_ref, c_ref, ids_ref, qn_ref, kn_ref, codes_ref, probs_ref):
    qm2 = q_ref[...]
    c = c_ref[0]
    ids = ids_ref[0]
    qn = qn_ref[0]
    kn = kn_ref[0]
    Q = qm2.shape[0]
    K = c.shape[0]
    nfull = K // _LANES
    inf = jnp.float32(jnp.inf)

    dots2 = jax.lax.dot_general(
        qm2, c, (((1,), (1,)), ((), ())),
        preferred_element_type=jnp.float32)
    d2 = jnp.maximum((qn + kn) + dots2, 0.0)


    depth = 3
    tiles = [d2[:, j * _LANES:(j + 1) * _LANES] for j in range(nfull)]
    if K % _LANES:
        tiles.append(jnp.concatenate(
            [d2[:, nfull * _LANES:],
             jnp.full((Q, _LANES - K % _LANES), inf, jnp.float32)], axis=1))
    s = [jnp.full((Q, _LANES), inf, jnp.float32) for _ in range(depth)]
    for v in tiles:
        ns = []
        for i in range(depth):
            ns.append(jnp.minimum(s[i], v))
            if i < depth - 1:
                v = jnp.maximum(s[i], v)
        s = ns

    cv = jnp.concatenate(s, axis=1)
    crow = jax.lax.broadcasted_iota(jnp.int32, (1, cv.shape[1]), 1)
    big_i = jnp.int32(2 ** 30)
    for _ in range(_TOPK - 1):
        m = jnp.min(cv, axis=1, keepdims=True)
        pm = jnp.min(jnp.where(cv == m, crow, big_i), axis=1, keepdims=True)
        cv = jnp.where(crow == pm, inf, cv)
    T2 = jnp.min(cv, axis=1, keepdims=True)

    t_d = jnp.sqrt(T2)
    a = t_d * t_d
    abits = jax.lax.bitcast_convert_type(a, jnp.int32)
    offs = jax.lax.broadcasted_iota(jnp.int32, (1, 33), 1) - jnp.int32(16)
    cand = jax.lax.bitcast_convert_type(abits + offs, jnp.float32)
    okc = jnp.sqrt(cand) <= t_d
    hi = jnp.max(jnp.where(okc, cand, -jnp.float32(jnp.inf)), axis=1,
                 keepdims=True)
    hi = jnp.maximum(hi, T2)
    succ_hi = jax.lax.bitcast_convert_type(
        jax.lax.bitcast_convert_type(hi, jnp.int32) + 1, jnp.float32)
    band_complete = jnp.all(jnp.sqrt(succ_hi) > t_d)

    lef = (d2 <= hi).astype(jnp.bfloat16)
    cions = jax.lax.broadcasted_iota(jnp.int32, (_NUM_CONCEPTS, 1), 0)
    onehot_t = (ids == cions).astype(jnp.bfloat16)
    counts_le = jax.lax.dot_general(
        lef, onehot_t, (((1,), (1,)), ((), ())),
        preferred_element_type=jnp.float32)
    n_le = jnp.sum(counts_le, axis=1, keepdims=True)
    all_exact = jnp.logical_and(
        jnp.all(n_le == jnp.float32(_TOPK)), band_complete)

    kiota_row = jax.lax.broadcasted_iota(jnp.int32, ids.shape, 1)
    pack_row = jnp.bitwise_or(jnp.left_shift(kiota_row, 5), ids)
    ciota = jax.lax.broadcasted_iota(jnp.int32, (Q, _NUM_CONCEPTS), 1)

    def fast_counts():
        return counts_le

    def exact_counts():
        dd = jnp.sqrt(d2)
        m = jnp.min(dd, axis=1, keepdims=True)
        counts = jnp.zeros((Q, _NUM_CONCEPTS), jnp.float32)
        for r in range(_TOPK):
            pm = jnp.min(jnp.where(dd == m, pack_row, big_i), axis=1,
                         keepdims=True)
            sel_id = jnp.bitwise_and(pm, jnp.int32(31))
            counts = counts + (sel_id == ciota).astype(jnp.float32)
            if r < _TOPK - 1:
                dd = jnp.where(pack_row == pm, inf, dd)
                m = jnp.min(dd, axis=1, keepdims=True)
        return counts

    counts = jax.lax.cond(all_exact, fast_counts, exact_counts)

    maxc = jnp.max(counts, axis=1, keepdims=True)
    code = jnp.min(jnp.where(counts == maxc, ciota, jnp.int32(_NUM_CONCEPTS)),
                   axis=1)
    codes_ref[0, 0, 0, :] = code.astype(jnp.float32)
    probs_ref[0, 0, 0, :] = maxc[:, 0] * (1.0 / _TOPK)


def kernel(slots, corpus_encs, corpus_ids):
    B, S, D = slots.shape
    nb, K, bs = corpus_encs.shape
    Q = B * S

    q_r = slots.reshape(Q, nb, bs)
    qm2 = slots.reshape(Q, nb * bs) * jnp.float32(-2.0)
    qn = jnp.transpose(jnp.sum(q_r * q_r, axis=-1), (1, 0))
    qn = qn.reshape(nb, Q, 1)
    kn = jnp.sum(corpus_encs * corpus_encs, axis=-1)

    ids3 = corpus_ids.reshape(nb, 1, K)
    kn3 = kn.reshape(nb, 1, K)

    qh = Q // 2
    codes, probs = pl.pallas_call(
        _knn_vote_body,
        grid=(nb, 2),
        in_specs=[
            pl.BlockSpec((qh, bs), lambda n, h: (h, n)),
            pl.BlockSpec((1, K, bs), lambda n, h: (n, 0, 0)),
            pl.BlockSpec((1, 1, K), lambda n, h: (n, 0, 0)),
            pl.BlockSpec((1, qh, 1), lambda n, h: (n, h, 0)),
            pl.BlockSpec((1, 1, K), lambda n, h: (n, 0, 0)),
        ],
        out_specs=[
            pl.BlockSpec((1, 1, 1, qh), lambda n, h: (n, h, 0, 0)),
            pl.BlockSpec((1, 1, 1, qh), lambda n, h: (n, h, 0, 0)),
        ],
        out_shape=[
            jax.ShapeDtypeStruct((nb, 2, 1, qh), jnp.float32),
            jax.ShapeDtypeStruct((nb, 2, 1, qh), jnp.float32),
        ],
        compiler_params=pltpu.CompilerParams(
            dimension_semantics=("arbitrary", "arbitrary"),
            vmem_limit_bytes=100 * 1024 * 1024,
        ),
    )(qm2, corpus_encs, ids3, qn, kn3)

    codes = jnp.transpose(codes.reshape(nb, Q), (1, 0)).reshape(B, S, nb)
    probs = jnp.transpose(probs.reshape(nb, Q), (1, 0)).reshape(B, S, nb)
    return codes, probs


# speedup vs baseline: 1.1728x; 1.1728x over previous
import jax
import jax.numpy as jnp
from jax.experimental import pallas as pl
from jax.experimental.pallas import tpu as pltpu

_NUM_CONCEPTS = 32
_TOPK = 5
_LANES = 128


def _knn_vote_body(q_ref, c_ref, ids_ref, qn_ref, kn_ref, codes_ref, probs_ref):
    qm2 = q_ref[...]
    c = c_ref[0]
    ids = ids_ref[0]
    qn = qn_ref[0]
    kn = kn_ref[0]
    Q = qm2.shape[0]
    K = c.shape[0]
    nfull = K // _LANES
    inf = jnp.float32(jnp.inf)

    dots2 = jax.lax.dot_general(
        qm2, c, (((1,), (1,)), ((), ())),
        preferred_element_type=jnp.float32)
    d2 = jnp.maximum((qn + kn) + dots2, 0.0)


    depth = 3
    tiles = [d2[:, j * _LANES:(j + 1) * _LANES] for j in range(nfull)]
    if K % _LANES:
        tiles.append(jnp.concatenate(
            [d2[:, nfull * _LANES:],
             jnp.full((Q, _LANES - K % _LANES), inf, jnp.float32)], axis=1))
    s = [jnp.full((Q, _LANES), inf, jnp.float32) for _ in range(depth)]
    for v in tiles:
        ns = []
        for i in range(depth):
            ns.append(jnp.minimum(s[i], v))
            if i < depth - 1:
                v = jnp.maximum(s[i], v)
        s = ns

    cv = jnp.concatenate(s, axis=1)
    crow = jax.lax.broadcasted_iota(jnp.int32, (1, cv.shape[1]), 1)
    big_i = jnp.int32(2 ** 30)
    for _ in range(_TOPK - 1):
        m = jnp.min(cv, axis=1, keepdims=True)
        pm = jnp.min(jnp.where(cv == m, crow, big_i), axis=1, keepdims=True)
        cv = jnp.where(crow == pm, inf, cv)
    T2 = jnp.min(cv, axis=1, keepdims=True)

    t_d = jnp.sqrt(T2)
    a = t_d * t_d
    abits = jax.lax.bitcast_convert_type(a, jnp.int32)
    offs = jax.lax.broadcasted_iota(jnp.int32, (1, 33), 1) - jnp.int32(16)
    cand = jax.lax.bitcast_convert_type(abits + offs, jnp.float32)
    okc = jnp.sqrt(cand) <= t_d
    hi = jnp.max(jnp.where(okc, cand, -jnp.float32(jnp.inf)), axis=1,
                 keepdims=True)
    hi = jnp.maximum(hi, T2)
    succ_hi = jax.lax.bitcast_convert_type(
        jax.lax.bitcast_convert_type(hi, jnp.int32) + 1, jnp.float32)
    band_complete = jnp.all(jnp.sqrt(succ_hi) > t_d)

    lef = (d2 <= hi).astype(jnp.bfloat16)
    cions = jax.lax.broadcasted_iota(jnp.int32, (_NUM_CONCEPTS, 1), 0)
    onehot_t = (ids == cions).astype(jnp.bfloat16)
    counts_le = jax.lax.dot_general(
        lef, onehot_t, (((1,), (1,)), ((), ())),
        preferred_element_type=jnp.float32)
    n_le = jnp.sum(counts_le, axis=1, keepdims=True)
    all_exact = jnp.logical_and(
        jnp.all(n_le == jnp.float32(_TOPK)), band_complete)

    kiota_row = jax.lax.broadcasted_iota(jnp.int32, ids.shape, 1)
    pack_row = jnp.bitwise_or(jnp.left_shift(kiota_row, 5), ids)
    ciota = jax.lax.broadcasted_iota(jnp.int32, (Q, _NUM_CONCEPTS), 1)

    def fast_counts():
        return counts_le

    def exact_counts():
        dd = jnp.sqrt(d2)
        m = jnp.min(dd, axis=1, keepdims=True)
        counts = jnp.zeros((Q, _NUM_CONCEPTS), jnp.float32)
        for r in range(_TOPK):
            pm = jnp.min(jnp.where(dd == m, pack_row, big_i), axis=1,
                         keepdims=True)
            sel_id = jnp.bitwise_and(pm, jnp.int32(31))
            counts = counts + (sel_id == ciota).astype(jnp.float32)
            if r < _TOPK - 1:
                dd = jnp.where(pack_row == pm, inf, dd)
                m = jnp.min(dd, axis=1, keepdims=True)
        return counts

    counts = jax.lax.cond(all_exact, fast_counts, exact_counts)

    maxc = jnp.max(counts, axis=1, keepdims=True)
    code = jnp.min(jnp.where(counts == maxc, ciota, jnp.int32(_NUM_CONCEPTS)),
                   axis=1)
    codes_ref[0, 0, :] = code.astype(jnp.float32)
    probs_ref[0, 0, :] = maxc[:, 0] * (1.0 / _TOPK)


def kernel(slots, corpus_encs, corpus_ids):
    B, S, D = slots.shape
    nb, K, bs = corpus_encs.shape
    Q = B * S

    q_r = slots.reshape(Q, nb, bs)
    qm2 = slots.reshape(Q, nb * bs) * jnp.float32(-2.0)
    qn = jnp.transpose(jnp.sum(q_r * q_r, axis=-1), (1, 0))
    qn = qn.reshape(nb, Q, 1)
    kn = jnp.sum(corpus_encs * corpus_encs, axis=-1)

    ids3 = corpus_ids.reshape(nb, 1, K)
    kn3 = kn.reshape(nb, 1, K)

    codes, probs = pl.pallas_call(
        _knn_vote_body,
        grid=(nb,),
        in_specs=[
            pl.BlockSpec((Q, bs), lambda n: (0, n)),
            pl.BlockSpec((1, K, bs), lambda n: (n, 0, 0)),
            pl.BlockSpec((1, 1, K), lambda n: (n, 0, 0)),
            pl.BlockSpec((1, Q, 1), lambda n: (n, 0, 0)),
            pl.BlockSpec((1, 1, K), lambda n: (n, 0, 0)),
        ],
        out_specs=[
            pl.BlockSpec((1, 1, Q), lambda n: (n, 0, 0)),
            pl.BlockSpec((1, 1, Q), lambda n: (n, 0, 0)),
        ],
        out_shape=[
            jax.ShapeDtypeStruct((nb, 1, Q), jnp.float32),
            jax.ShapeDtypeStruct((nb, 1, Q), jnp.float32),
        ],
        compiler_params=pltpu.CompilerParams(
            dimension_semantics=("arbitrary",),
            vmem_limit_bytes=100 * 1024 * 1024,
        ),
    )(qm2, corpus_encs, ids3, qn, kn3)

    codes = jnp.transpose(codes.reshape(nb, Q), (1, 0)).reshape(B, S, nb)
    probs = jnp.transpose(probs.reshape(nb, Q), (1, 0)).reshape(B, S, nb)
    return codes, probs
